# Initial kernel scaffold; baseline (speedup 1.0000x reference)
import jax
import jax.numpy as jnp
from jax.experimental import pallas as pl


def kernel(x, edge_index, W1, b1, W2, b2):
    raise NotImplementedError("write your pallas kernel here")


# speedup vs baseline: 16.0057x; 16.0057x over previous
import functools

import jax
import jax.numpy as jnp
from jax import lax
from jax.experimental import pallas as pl
from jax.experimental.pallas import tpu as pltpu
from jax.experimental.pallas import tpu_sc as plsc

N = 10000
NPAD = 10240
F_IN = 128
H = 64
C = 40
CP = 48
E = 320000
NW = 32
EPW = 10240
CH = 128
NCH = EPW // CH
EPAD = NW * EPW
TPS = 16
RPW = NPAD // TPS
DEGW = 16

_mesh = plsc.VectorSubcoreMesh(core_axis_name="c", subcore_axis_name="s")
_sc_params = pltpu.CompilerParams(use_tc_tiling_on_sc=False)


def _fill_rows(buf, nrows, width, value):
    v = jnp.full((16,), value, jnp.float32)

    @pl.loop(0, nrows)
    def _(r):
        for cc in range(0, width, 16):
            buf[r, pl.ds(cc, 16)] = v


@functools.partial(
    pl.kernel,
    out_type=jax.ShapeDtypeStruct((2, NPAD, DEGW), jnp.float32),
    mesh=_mesh,
    compiler_params=_sc_params,
    scratch_types=[
        pltpu.VMEM((NCH, CH), jnp.int32),
        pltpu.VMEM((CH, DEGW), jnp.float32),
        pltpu.VMEM_SHARED((NPAD, DEGW), jnp.float32),
    ],
)
def _deg_kernel(dst_hbm, out_hbm, dst_v, ones_v, acc_sh):
    c = lax.axis_index("c")
    s = lax.axis_index("s")
    w = c * TPS + s
    pltpu.sync_copy(dst_hbm.at[w], dst_v)
    _fill_rows(ones_v, CH, DEGW, 0.0)

    @pl.loop(0, RPW // CH)
    def _(k):
        pltpu.sync_copy(ones_v, acc_sh.at[pl.ds(s * RPW + k * CH, CH)])

    _fill_rows(ones_v, CH, DEGW, 1.0)
    plsc.subcore_barrier()

    @pl.loop(0, NCH)
    def _(j):
        pltpu.sync_copy(ones_v, acc_sh.at[dst_v.at[j]], add=True)

    plsc.subcore_barrier()
    pltpu.sync_copy(acc_sh.at[pl.ds(s * RPW, RPW)],
                    out_hbm.at[c, pl.ds(s * RPW, RPW)])


def _make_agg(D):
    @functools.partial(
        pl.kernel,
        out_type=jax.ShapeDtypeStruct((2, NPAD, D), jnp.float32),
        mesh=_mesh,
        compiler_params=_sc_params,
        scratch_types=[
            pltpu.VMEM((NCH, CH), jnp.int32),
            pltpu.VMEM((NCH, CH), jnp.int32),
            pltpu.VMEM((CH, D), jnp.float32),
            pltpu.VMEM_SHARED((NPAD, D), jnp.float32),
        ],
    )
    def _agg(h_hbm, src_hbm, dst_hbm, out_hbm, src_v, dst_v, rows_v, acc_sh):
        c = lax.axis_index("c")
        s = lax.axis_index("s")
        w = c * TPS + s
        pltpu.sync_copy(src_hbm.at[w], src_v)
        pltpu.sync_copy(dst_hbm.at[w], dst_v)
        _fill_rows(rows_v, CH, D, 0.0)

        @pl.loop(0, RPW // CH)
        def _(k):
            pltpu.sync_copy(rows_v, acc_sh.at[pl.ds(s * RPW + k * CH, CH)])

        plsc.subcore_barrier()

        @pl.loop(0, NCH)
        def _(j):
            pltpu.sync_copy(h_hbm.at[src_v.at[j]], rows_v)
            pltpu.sync_copy(rows_v, acc_sh.at[dst_v.at[j]], add=True)

        plsc.subcore_barrier()
        pltpu.sync_copy(acc_sh.at[pl.ds(s * RPW, RPW)],
                        out_hbm.at[c, pl.ds(s * RPW, RPW)])

    return _agg


_agg_h = _make_agg(H)
_agg_c = _make_agg(CP)


def _dinv(degp_ref):
    deg = degp_ref[0, :, 0:1] + degp_ref[1, :, 0:1] + 1.0
    return lax.rsqrt(deg)


def _tc_mm1(x_ref, w1_ref, o_ref):
    o_ref[...] = jnp.dot(x_ref[...], w1_ref[...],
                         preferred_element_type=jnp.float32)


def _tc_scale(degp_ref, h_ref, o_ref):
    o_ref[...] = _dinv(degp_ref) * h_ref[...]


def _tc_mid(p_ref, hp_ref, degp_ref, w2_ref, b1_ref, o_ref):
    dinv = _dinv(degp_ref)
    a = dinv * (p_ref[0] + p_ref[1] + hp_ref[...]) + b1_ref[...]
    a = jnp.maximum(a, 0.0)
    o_ref[...] = dinv * jnp.dot(a, w2_ref[...],
                                preferred_element_type=jnp.float32)


def _tc_out(p_ref, hp_ref, degp_ref, b2_ref, o_ref):
    dinv = _dinv(degp_ref)
    o = dinv * (p_ref[0] + p_ref[1] + hp_ref[...]) + b2_ref[...]
    col = lax.broadcasted_iota(jnp.int32, o.shape, 1)
    valid = col < C
    neg = jnp.float32(-3.0e38)
    m = jnp.max(jnp.where(valid, o, neg), axis=1, keepdims=True)
    ssum = jnp.sum(jnp.where(valid, jnp.exp(o - m), 0.0), axis=1,
                   keepdims=True)
    o_ref[...] = o - m - jnp.log(ssum)


def _call(fn, out_shape, *args):
    return pl.pallas_call(
        fn, out_shape=jax.ShapeDtypeStruct(out_shape, jnp.float32))(*args)


def kernel(x, edge_index, W1, b1, W2, b2):
    src = edge_index[0].astype(jnp.int32)
    dst = edge_index[1].astype(jnp.int32)
    pad = jnp.full((EPAD - E,), N, jnp.int32)
    srcp = jnp.concatenate([src, pad]).reshape(NW, NCH, CH)
    dstp = jnp.concatenate([dst, pad]).reshape(NW, NCH, CH)

    xp = jnp.pad(x, ((0, NPAD - N), (0, 0)))
    w2p = jnp.pad(W2, ((0, 0), (0, CP - C)))
    b1r = b1.reshape(1, H)
    b2r = jnp.pad(b2, (0, CP - C)).reshape(1, CP)

    degp = _deg_kernel(dstp)
    h1 = _call(_tc_mm1, (NPAD, H), xp, W1)
    h1p = _call(_tc_scale, (NPAD, H), degp, h1)
    p1 = _agg_h(h1p, srcp, dstp)
    h2p = _call(_tc_mid, (NPAD, CP), p1, h1p, degp, w2p, b1r)
    p2 = _agg_c(h2p, srcp, dstp)
    out = _call(_tc_out, (NPAD, CP), p2, h2p, degp, b2r)
    return out[:N, :C]


# speedup vs baseline: 17.3173x; 1.0819x over previous
import functools

import jax
import jax.numpy as jnp
from jax import lax
from jax.experimental import pallas as pl
from jax.experimental.pallas import tpu as pltpu
from jax.experimental.pallas import tpu_sc as plsc

N = 10000
NPAD = 10240
F_IN = 128
H = 64
C = 40
CP = 48
E = 320000
NW = 32
EPW = 10240
CH = 128
NCH = EPW // CH
EPAD = NW * EPW
TPS = 16
RPW = NPAD // TPS
DEGW = 16

_mesh = plsc.VectorSubcoreMesh(core_axis_name="c", subcore_axis_name="s")
_sc_params = pltpu.CompilerParams(use_tc_tiling_on_sc=False)


def _fill_rows(buf, nrows, width, value):
    v = jnp.full((16,), value, jnp.float32)

    @pl.loop(0, nrows)
    def _(r):
        for cc in range(0, width, 16):
            buf[r, pl.ds(cc, 16)] = v


@functools.partial(
    pl.kernel,
    out_type=jax.ShapeDtypeStruct((2, NPAD, DEGW), jnp.float32),
    mesh=_mesh,
    compiler_params=_sc_params,
    scratch_types=[
        pltpu.VMEM((NCH, CH), jnp.int32),
        pltpu.VMEM((CH, DEGW), jnp.float32),
        pltpu.VMEM_SHARED((NPAD, DEGW), jnp.float32),
    ],
)
def _deg_kernel(dst_hbm, out_hbm, dst_v, ones_v, acc_sh):
    c = lax.axis_index("c")
    s = lax.axis_index("s")
    w = c * TPS + s
    pltpu.sync_copy(dst_hbm.at[w], dst_v)
    _fill_rows(ones_v, CH, DEGW, 0.0)

    @pl.loop(0, RPW // CH)
    def _(k):
        pltpu.sync_copy(ones_v, acc_sh.at[pl.ds(s * RPW + k * CH, CH)])

    _fill_rows(ones_v, CH, DEGW, 1.0)
    plsc.subcore_barrier()

    @pl.loop(0, NCH)
    def _(j):
        pltpu.sync_copy(ones_v, acc_sh.at[dst_v.at[j]], add=True)

    plsc.subcore_barrier()
    pltpu.sync_copy(acc_sh.at[pl.ds(s * RPW, RPW)],
                    out_hbm.at[c, pl.ds(s * RPW, RPW)])


def _make_agg(D):
    @functools.partial(
        pl.kernel,
        out_type=jax.ShapeDtypeStruct((2, NPAD, D), jnp.float32),
        mesh=_mesh,
        compiler_params=_sc_params,
        scratch_types=[
            pltpu.VMEM((NCH, CH), jnp.int32),
            pltpu.VMEM((NCH, CH), jnp.int32),
            pltpu.VMEM((CH, D), jnp.float32),
            pltpu.VMEM((CH, D), jnp.float32),
            pltpu.VMEM_SHARED((NPAD, D), jnp.float32),
            pltpu.SemaphoreType.DMA,
            pltpu.SemaphoreType.DMA,
            pltpu.SemaphoreType.DMA,
            pltpu.SemaphoreType.DMA,
        ],
    )
    def _agg(h_hbm, src_hbm, dst_hbm, out_hbm, src_v, dst_v, buf0, buf1,
             acc_sh, g0, g1, s0, s1):
        c = lax.axis_index("c")
        s = lax.axis_index("s")
        w = c * TPS + s
        pltpu.sync_copy(src_hbm.at[w], src_v)
        pltpu.sync_copy(dst_hbm.at[w], dst_v)
        _fill_rows(buf0, CH, D, 0.0)

        @pl.loop(0, RPW // CH)
        def _(k):
            pltpu.sync_copy(buf0, acc_sh.at[pl.ds(s * RPW + k * CH, CH)])

        plsc.subcore_barrier()

        pltpu.async_copy(h_hbm.at[src_v.at[0]], buf0, g0)

        @pl.loop(0, NCH, step=2)
        def _(j):
            pltpu.make_async_copy(h_hbm.at[src_v.at[j]], buf0, g0).wait()
            sc0 = pltpu.async_copy(buf0, acc_sh.at[dst_v.at[j]], s0,
                                   add=True)
            gb1 = pltpu.async_copy(h_hbm.at[src_v.at[j + 1]], buf1, g1)
            sc0.wait()
            gb1.wait()
            sc1 = pltpu.async_copy(buf1, acc_sh.at[dst_v.at[j + 1]], s1,
                                   add=True)

            @pl.when(j + 2 < NCH)
            def _():
                pltpu.async_copy(h_hbm.at[src_v.at[j + 2]], buf0, g0)

            sc1.wait()

        plsc.subcore_barrier()
        pltpu.sync_copy(acc_sh.at[pl.ds(s * RPW, RPW)],
                        out_hbm.at[c, pl.ds(s * RPW, RPW)])

    return _agg


_agg_h = _make_agg(H)
_agg_c = _make_agg(CP)


def _dinv(degp_ref):
    deg = degp_ref[0, :, 0:1] + degp_ref[1, :, 0:1] + 1.0
    return lax.rsqrt(deg)


def _tc_mm1(x_ref, w1_ref, o_ref):
    o_ref[...] = jnp.dot(x_ref[...], w1_ref[...],
                         preferred_element_type=jnp.float32)


def _tc_scale(degp_ref, h_ref, o_ref):
    o_ref[...] = _dinv(degp_ref) * h_ref[...]


def _tc_mid(p_ref, hp_ref, degp_ref, w2_ref, b1_ref, o_ref):
    dinv = _dinv(degp_ref)
    a = dinv * (p_ref[0] + p_ref[1] + hp_ref[...]) + b1_ref[...]
    a = jnp.maximum(a, 0.0)
    o_ref[...] = dinv * jnp.dot(a, w2_ref[...],
                                preferred_element_type=jnp.float32)


def _tc_out(p_ref, hp_ref, degp_ref, b2_ref, o_ref):
    dinv = _dinv(degp_ref)
    o = dinv * (p_ref[0] + p_ref[1] + hp_ref[...]) + b2_ref[...]
    col = lax.broadcasted_iota(jnp.int32, o.shape, 1)
    valid = col < C
    neg = jnp.float32(-3.0e38)
    m = jnp.max(jnp.where(valid, o, neg), axis=1, keepdims=True)
    ssum = jnp.sum(jnp.where(valid, jnp.exp(o - m), 0.0), axis=1,
                   keepdims=True)
    o_ref[...] = o - m - jnp.log(ssum)


def _call(fn, out_shape, *args):
    return pl.pallas_call(
        fn, out_shape=jax.ShapeDtypeStruct(out_shape, jnp.float32))(*args)


def kernel(x, edge_index, W1, b1, W2, b2):
    src = edge_index[0].astype(jnp.int32)
    dst = edge_index[1].astype(jnp.int32)
    pad = jnp.full((EPAD - E,), N, jnp.int32)
    srcp = jnp.concatenate([src, pad]).reshape(NW, NCH, CH)
    dstp = jnp.concatenate([dst, pad]).reshape(NW, NCH, CH)

    xp = jnp.pad(x, ((0, NPAD - N), (0, 0)))
    w2p = jnp.pad(W2, ((0, 0), (0, CP - C)))
    b1r = b1.reshape(1, H)
    b2r = jnp.pad(b2, (0, CP - C)).reshape(1, CP)

    degp = _deg_kernel(dstp)
    h1 = _call(_tc_mm1, (NPAD, H), xp, W1)
    h1p = _call(_tc_scale, (NPAD, H), degp, h1)
    p1 = _agg_h(h1p, srcp, dstp)
    h2p = _call(_tc_mid, (NPAD, CP), p1, h1p, degp, w2p, b1r)
    p2 = _agg_c(h2p, srcp, dstp)
    out = _call(_tc_out, (NPAD, CP), p2, h2p, degp, b2r)
    return out[:N, :C]


# speedup vs baseline: 36.5838x; 2.1126x over previous
import functools

import jax
import jax.numpy as jnp
from jax import lax
from jax.experimental import pallas as pl
from jax.experimental.pallas import tpu as pltpu
from jax.experimental.pallas import tpu_sc as plsc

N = 10000
NPAD = 10240
F_IN = 128
H = 64
C = 40
CP = 48
E = 320000
NW = 32
EPW = 10240
CH = 128
NCH = EPW // CH
EPAD = NW * EPW
TPS = 16
RPW = NPAD // TPS
DEGW = 16

_mesh = plsc.VectorSubcoreMesh(core_axis_name="c", subcore_axis_name="s")
_sc_params = pltpu.CompilerParams(use_tc_tiling_on_sc=False)


def _fill_rows(buf, nrows, width, value):
    v = jnp.full((16,), value, jnp.float32)

    @pl.loop(0, nrows)
    def _(r):
        for cc in range(0, width, 16):
            buf[r, pl.ds(cc, 16)] = v


@functools.partial(
    pl.kernel,
    out_type=jax.ShapeDtypeStruct((2, NPAD, DEGW), jnp.float32),
    mesh=_mesh,
    compiler_params=_sc_params,
    scratch_types=[
        pltpu.VMEM((NCH, CH), jnp.int32),
        pltpu.VMEM((CH, DEGW), jnp.float32),
        pltpu.VMEM_SHARED((NPAD, DEGW), jnp.float32),
    ],
)
def _deg_kernel(dst_hbm, out_hbm, dst_v, ones_v, acc_sh):
    c = lax.axis_index("c")
    s = lax.axis_index("s")
    w = c * TPS + s
    pltpu.sync_copy(dst_hbm.at[w], dst_v)
    _fill_rows(ones_v, CH, DEGW, 0.0)

    @pl.loop(0, RPW // CH)
    def _(k):
        pltpu.sync_copy(ones_v, acc_sh.at[pl.ds(s * RPW + k * CH, CH)])

    _fill_rows(ones_v, CH, DEGW, 1.0)
    plsc.subcore_barrier()

    @pl.loop(0, NCH)
    def _(j):
        pltpu.sync_copy(ones_v, acc_sh.at[dst_v.at[j]], add=True)

    plsc.subcore_barrier()
    pltpu.sync_copy(acc_sh.at[pl.ds(s * RPW, RPW)],
                    out_hbm.at[c, pl.ds(s * RPW, RPW)])


def _make_agg(D):
    @functools.partial(
        pl.kernel,
        out_type=jax.ShapeDtypeStruct((2, NPAD, D), jnp.float32),
        mesh=_mesh,
        compiler_params=_sc_params,
        scratch_types=[
            pltpu.VMEM((NCH, CH), jnp.int32),
            pltpu.VMEM((NCH, CH), jnp.int32),
            pltpu.VMEM((CH, D), jnp.float32),
            pltpu.VMEM((CH, D), jnp.float32),
            pltpu.VMEM_SHARED((NPAD, D), jnp.float32),
            pltpu.VMEM_SHARED((NPAD, D), jnp.float32),
            pltpu.SemaphoreType.DMA,
            pltpu.SemaphoreType.DMA,
            pltpu.SemaphoreType.DMA,
            pltpu.SemaphoreType.DMA,
        ],
    )
    def _agg(h_hbm, src_hbm, dst_hbm, out_hbm, src_v, dst_v, buf0, buf1,
             acc_sh, h_sh, g0, g1, s0, s1):
        c = lax.axis_index("c")
        s = lax.axis_index("s")
        w = c * TPS + s
        hst = pltpu.async_copy(h_hbm.at[pl.ds(s * RPW, RPW)],
                               h_sh.at[pl.ds(s * RPW, RPW)], g1)
        pltpu.sync_copy(src_hbm.at[w], src_v)
        pltpu.sync_copy(dst_hbm.at[w], dst_v)
        _fill_rows(buf0, CH, D, 0.0)

        @pl.loop(0, RPW // CH)
        def _(k):
            pltpu.sync_copy(buf0, acc_sh.at[pl.ds(s * RPW + k * CH, CH)])

        hst.wait()
        plsc.subcore_barrier()

        pltpu.async_copy(h_sh.at[src_v.at[0]], buf0, g0)

        @pl.loop(0, NCH, step=2)
        def _(j):
            pltpu.make_async_copy(h_sh.at[src_v.at[j]], buf0, g0).wait()
            sc0 = pltpu.async_copy(buf0, acc_sh.at[dst_v.at[j]], s0,
                                   add=True)
            gb1 = pltpu.async_copy(h_sh.at[src_v.at[j + 1]], buf1, g1)
            sc0.wait()
            gb1.wait()
            sc1 = pltpu.async_copy(buf1, acc_sh.at[dst_v.at[j + 1]], s1,
                                   add=True)

            @pl.when(j + 2 < NCH)
            def _():
                pltpu.async_copy(h_sh.at[src_v.at[j + 2]], buf0, g0)

            sc1.wait()

        plsc.subcore_barrier()
        pltpu.sync_copy(acc_sh.at[pl.ds(s * RPW, RPW)],
                        out_hbm.at[c, pl.ds(s * RPW, RPW)])

    return _agg


_agg_h = _make_agg(H)
_agg_c = _make_agg(CP)


def _dinv(degp_ref):
    deg = degp_ref[0, :, 0:1] + degp_ref[1, :, 0:1] + 1.0
    return lax.rsqrt(deg)


def _tc_mm1(x_ref, w1_ref, o_ref):
    o_ref[...] = jnp.dot(x_ref[...], w1_ref[...],
                         preferred_element_type=jnp.float32)


def _tc_scale(degp_ref, h_ref, o_ref):
    o_ref[...] = _dinv(degp_ref) * h_ref[...]


def _tc_mid(p_ref, hp_ref, degp_ref, w2_ref, b1_ref, o_ref):
    dinv = _dinv(degp_ref)
    a = dinv * (p_ref[0] + p_ref[1] + hp_ref[...]) + b1_ref[...]
    a = jnp.maximum(a, 0.0)
    o_ref[...] = dinv * jnp.dot(a, w2_ref[...],
                                preferred_element_type=jnp.float32)


def _tc_out(p_ref, hp_ref, degp_ref, b2_ref, o_ref):
    dinv = _dinv(degp_ref)
    o = dinv * (p_ref[0] + p_ref[1] + hp_ref[...]) + b2_ref[...]
    col = lax.broadcasted_iota(jnp.int32, o.shape, 1)
    valid = col < C
    neg = jnp.float32(-3.0e38)
    m = jnp.max(jnp.where(valid, o, neg), axis=1, keepdims=True)
    ssum = jnp.sum(jnp.where(valid, jnp.exp(o - m), 0.0), axis=1,
                   keepdims=True)
    o_ref[...] = o - m - jnp.log(ssum)


def _call(fn, out_shape, *args):
    return pl.pallas_call(
        fn, out_shape=jax.ShapeDtypeStruct(out_shape, jnp.float32))(*args)


def kernel(x, edge_index, W1, b1, W2, b2):
    src = edge_index[0].astype(jnp.int32)
    dst = edge_index[1].astype(jnp.int32)
    pad = jnp.full((EPAD - E,), N, jnp.int32)
    srcp = jnp.concatenate([src, pad]).reshape(NW, NCH, CH)
    dstp = jnp.concatenate([dst, pad]).reshape(NW, NCH, CH)

    xp = jnp.pad(x, ((0, NPAD - N), (0, 0)))
    w2p = jnp.pad(W2, ((0, 0), (0, CP - C)))
    b1r = b1.reshape(1, H)
    b2r = jnp.pad(b2, (0, CP - C)).reshape(1, CP)

    degp = _deg_kernel(dstp)
    h1 = _call(_tc_mm1, (NPAD, H), xp, W1)
    h1p = _call(_tc_scale, (NPAD, H), degp, h1)
    p1 = _agg_h(h1p, srcp, dstp)
    h2p = _call(_tc_mid, (NPAD, CP), p1, h1p, degp, w2p, b1r)
    p2 = _agg_c(h2p, srcp, dstp)
    out = _call(_tc_out, (NPAD, CP), p2, h2p, degp, b2r)
    return out[:N, :C]


# speedup vs baseline: 36.5942x; 1.0003x over previous
import functools

import jax
import jax.numpy as jnp
from jax import lax
from jax.experimental import pallas as pl
from jax.experimental.pallas import tpu as pltpu
from jax.experimental.pallas import tpu_sc as plsc

N = 10000
NPAD = 10240
F_IN = 128
H = 64
C = 40
CP = 48
E = 320000
NW = 32
EPW = 10240
CH = 128
NCH = EPW // CH
EPAD = NW * EPW
TPS = 16
RPW = NPAD // TPS
DEGW = 16

_mesh = plsc.VectorSubcoreMesh(core_axis_name="c", subcore_axis_name="s")
_sc_params = pltpu.CompilerParams(use_tc_tiling_on_sc=False)


def _fill_rows(buf, nrows, width, value):
    v = jnp.full((16,), value, jnp.float32)

    @pl.loop(0, nrows)
    def _(r):
        for cc in range(0, width, 16):
            buf[r, pl.ds(cc, 16)] = v


@functools.partial(
    pl.kernel,
    out_type=jax.ShapeDtypeStruct((2, NPAD, DEGW), jnp.float32),
    mesh=_mesh,
    compiler_params=_sc_params,
    scratch_types=[
        pltpu.VMEM((NCH, CH), jnp.int32),
        pltpu.VMEM((CH, DEGW), jnp.float32),
        pltpu.VMEM_SHARED((NPAD, DEGW), jnp.float32),
        pltpu.SemaphoreType.DMA,
    ],
)
def _deg_kernel(dst_hbm, out_hbm, dst_v, ones_v, acc_sh, sem):
    c = lax.axis_index("c")
    s = lax.axis_index("s")
    w = c * TPS + s
    pltpu.sync_copy(dst_hbm.at[w], dst_v)
    _fill_rows(ones_v, CH, DEGW, 0.0)

    @pl.loop(0, RPW // CH)
    def _(k):
        pltpu.sync_copy(ones_v, acc_sh.at[pl.ds(s * RPW + k * CH, CH)])

    _fill_rows(ones_v, CH, DEGW, 1.0)
    plsc.subcore_barrier()

    @pl.loop(0, NCH, step=8)
    def _(j):
        for k in range(8):
            pltpu.async_copy(ones_v, acc_sh.at[dst_v.at[j + k]], sem,
                             add=True)
        for k in range(8):
            pltpu.make_async_copy(ones_v, acc_sh.at[dst_v.at[j + k]],
                                  sem).wait()

    plsc.subcore_barrier()
    pltpu.sync_copy(acc_sh.at[pl.ds(s * RPW, RPW)],
                    out_hbm.at[c, pl.ds(s * RPW, RPW)])


def _make_agg(D):
    @functools.partial(
        pl.kernel,
        out_type=jax.ShapeDtypeStruct((2, NPAD, D), jnp.float32),
        mesh=_mesh,
        compiler_params=_sc_params,
        scratch_types=[
            pltpu.VMEM((NCH, CH), jnp.int32),
            pltpu.VMEM((NCH, CH), jnp.int32),
            pltpu.VMEM((CH, D), jnp.float32),
            pltpu.VMEM((CH, D), jnp.float32),
            pltpu.VMEM_SHARED((NPAD, D), jnp.float32),
            pltpu.VMEM_SHARED((NPAD, D), jnp.float32),
            pltpu.SemaphoreType.DMA,
            pltpu.SemaphoreType.DMA,
            pltpu.SemaphoreType.DMA,
            pltpu.SemaphoreType.DMA,
        ],
    )
    def _agg(h_hbm, src_hbm, dst_hbm, out_hbm, src_v, dst_v, buf0, buf1,
             acc_sh, h_sh, g0, g1, s0, s1):
        c = lax.axis_index("c")
        s = lax.axis_index("s")
        w = c * TPS + s
        hst = pltpu.async_copy(h_hbm.at[pl.ds(s * RPW, RPW)],
                               h_sh.at[pl.ds(s * RPW, RPW)], g1)
        pltpu.sync_copy(src_hbm.at[w], src_v)
        pltpu.sync_copy(dst_hbm.at[w], dst_v)
        _fill_rows(buf0, CH, D, 0.0)

        @pl.loop(0, RPW // CH)
        def _(k):
            pltpu.sync_copy(buf0, acc_sh.at[pl.ds(s * RPW + k * CH, CH)])

        hst.wait()
        plsc.subcore_barrier()

        pltpu.async_copy(h_sh.at[src_v.at[0]], buf0, g0)

        @pl.loop(0, NCH, step=2)
        def _(j):
            pltpu.make_async_copy(h_sh.at[src_v.at[j]], buf0, g0).wait()
            sc0 = pltpu.async_copy(buf0, acc_sh.at[dst_v.at[j]], s0,
                                   add=True)
            gb1 = pltpu.async_copy(h_sh.at[src_v.at[j + 1]], buf1, g1)
            sc0.wait()
            gb1.wait()
            sc1 = pltpu.async_copy(buf1, acc_sh.at[dst_v.at[j + 1]], s1,
                                   add=True)

            @pl.when(j + 2 < NCH)
            def _():
                pltpu.async_copy(h_sh.at[src_v.at[j + 2]], buf0, g0)

            sc1.wait()

        plsc.subcore_barrier()
        pltpu.sync_copy(acc_sh.at[pl.ds(s * RPW, RPW)],
                        out_hbm.at[c, pl.ds(s * RPW, RPW)])

    return _agg


_agg_h = _make_agg(H)
_agg_c = _make_agg(CP)


def _dinv(degp_ref):
    deg = degp_ref[0, :, 0:1] + degp_ref[1, :, 0:1] + 1.0
    return lax.rsqrt(deg)


def _tc_mm1(x_ref, w1_ref, o_ref):
    o_ref[...] = jnp.dot(x_ref[...], w1_ref[...],
                         preferred_element_type=jnp.float32)


def _tc_scale(degp_ref, h_ref, o_ref):
    o_ref[...] = _dinv(degp_ref) * h_ref[...]


def _tc_mid(p_ref, hp_ref, degp_ref, w2_ref, b1_ref, o_ref):
    dinv = _dinv(degp_ref)
    a = dinv * (p_ref[0] + p_ref[1] + hp_ref[...]) + b1_ref[...]
    a = jnp.maximum(a, 0.0)
    o_ref[...] = dinv * jnp.dot(a, w2_ref[...],
                                preferred_element_type=jnp.float32)


def _tc_out(p_ref, hp_ref, degp_ref, b2_ref, o_ref):
    dinv = _dinv(degp_ref)
    o = dinv * (p_ref[0] + p_ref[1] + hp_ref[...]) + b2_ref[...]
    col = lax.broadcasted_iota(jnp.int32, o.shape, 1)
    valid = col < C
    neg = jnp.float32(-3.0e38)
    m = jnp.max(jnp.where(valid, o, neg), axis=1, keepdims=True)
    ssum = jnp.sum(jnp.where(valid, jnp.exp(o - m), 0.0), axis=1,
                   keepdims=True)
    o_ref[...] = (o - m - jnp.log(ssum))[:N, :C]


def _call(fn, out_shape, *args):
    return pl.pallas_call(
        fn, out_shape=jax.ShapeDtypeStruct(out_shape, jnp.float32))(*args)


def kernel(x, edge_index, W1, b1, W2, b2):
    src = edge_index[0].astype(jnp.int32)
    dst = edge_index[1].astype(jnp.int32)
    pad = jnp.full((EPAD - E,), N, jnp.int32)
    srcp = jnp.concatenate([src, pad]).reshape(NW, NCH, CH)
    dstp = jnp.concatenate([dst, pad]).reshape(NW, NCH, CH)

    xp = jnp.pad(x, ((0, NPAD - N), (0, 0)))
    w2p = jnp.pad(W2, ((0, 0), (0, CP - C)))
    b1r = b1.reshape(1, H)
    b2r = jnp.pad(b2, (0, CP - C)).reshape(1, CP)

    degp = _deg_kernel(dstp)
    h1 = _call(_tc_mm1, (NPAD, H), xp, W1)
    h1p = _call(_tc_scale, (NPAD, H), degp, h1)
    p1 = _agg_h(h1p, srcp, dstp)
    h2p = _call(_tc_mid, (NPAD, CP), p1, h1p, degp, w2p, b1r)
    p2 = _agg_c(h2p, srcp, dstp)
    return _call(_tc_out, (N, C), p2, h2p, degp, b2r)


# speedup vs baseline: 37.7265x; 1.0309x over previous
import dataclasses
import functools

import jax
import jax.numpy as jnp
from jax import lax
from jax.experimental import pallas as pl
from jax.experimental.pallas import tpu as pltpu
from jax.experimental.pallas import tpu_sc as plsc

N = 10000
NPAD = 10240
F_IN = 128
H = 64
C = 40
CP = 48
E = 320000
NW = 32
EPW = 10240
CH = 128
NCH = EPW // CH
EPAD = NW * EPW
TPS = 16
RPW = NPAD // TPS
DEGW = 16

_mesh = plsc.VectorSubcoreMesh(core_axis_name="c", subcore_axis_name="s")
_sc_params = pltpu.CompilerParams(use_tc_tiling_on_sc=False)
if "needs_layout_passes" in pltpu.CompilerParams.__dataclass_fields__:
    _sc_params = dataclasses.replace(_sc_params, needs_layout_passes=False)


def _fill_rows(buf, nrows, width, value):
    v = jnp.full((16,), value, jnp.float32)

    @pl.loop(0, nrows)
    def _(r):
        for cc in range(0, width, 16):
            buf[r, pl.ds(cc, 16)] = v


@functools.partial(
    pl.kernel,
    out_type=jax.ShapeDtypeStruct((2, NPAD), jnp.float32),
    mesh=_mesh,
    compiler_params=_sc_params,
    scratch_types=[
        pltpu.VMEM((NCH, CH), jnp.int32),
        pltpu.VMEM((CH, DEGW), jnp.float32),
        pltpu.VMEM((RPW, DEGW), jnp.float32),
        pltpu.VMEM((RPW,), jnp.float32),
        pltpu.VMEM_SHARED((NPAD, DEGW), jnp.float32),
        pltpu.SemaphoreType.DMA,
    ],
)
def _deg_kernel(dst_hbm, out_hbm, dst_v, ones_v, reg_v, cmp_v, acc_sh, sem):
    c = lax.axis_index("c")
    s = lax.axis_index("s")
    w = c * TPS + s
    pltpu.sync_copy(dst_hbm.at[w], dst_v)
    _fill_rows(ones_v, CH, DEGW, 0.0)

    @pl.loop(0, RPW // CH)
    def _(k):
        pltpu.sync_copy(ones_v, acc_sh.at[pl.ds(s * RPW + k * CH, CH)])

    _fill_rows(ones_v, CH, DEGW, 1.0)
    plsc.subcore_barrier()

    @pl.loop(0, NCH, step=8)
    def _(j):
        for k in range(8):
            pltpu.async_copy(ones_v, acc_sh.at[dst_v.at[j + k]], sem,
                             add=True)
        for k in range(8):
            pltpu.make_async_copy(ones_v, acc_sh.at[dst_v.at[j + k]],
                                  sem).wait()

    plsc.subcore_barrier()
    pltpu.sync_copy(acc_sh.at[pl.ds(s * RPW, RPW)], reg_v)
    lane = jnp.arange(16, dtype=jnp.int32)
    zero16 = jnp.zeros((16,), jnp.int32)

    @pl.loop(0, RPW // 16)
    def _(j):
        vals = plsc.load_gather(reg_v, [j * 16 + lane, zero16])
        cmp_v[pl.ds(j * 16, 16)] = vals

    pltpu.sync_copy(cmp_v, out_hbm.at[c, pl.ds(s * RPW, RPW)])


def _make_agg(D):
    @functools.partial(
        pl.kernel,
        out_type=jax.ShapeDtypeStruct((2, NPAD, D), jnp.float32),
        mesh=_mesh,
        compiler_params=_sc_params,
        scratch_types=[
            pltpu.VMEM((NCH, CH), jnp.int32),
            pltpu.VMEM((NCH, CH), jnp.int32),
            pltpu.VMEM((CH, D), jnp.float32),
            pltpu.VMEM((CH, D), jnp.float32),
            pltpu.VMEM_SHARED((NPAD, D), jnp.float32),
            pltpu.VMEM_SHARED((NPAD, D), jnp.float32),
            pltpu.SemaphoreType.DMA,
            pltpu.SemaphoreType.DMA,
            pltpu.SemaphoreType.DMA,
            pltpu.SemaphoreType.DMA,
        ],
    )
    def _agg(h_hbm, src_hbm, dst_hbm, out_hbm, src_v, dst_v, buf0, buf1,
             acc_sh, h_sh, g0, g1, s0, s1):
        c = lax.axis_index("c")
        s = lax.axis_index("s")
        w = c * TPS + s
        hst = pltpu.async_copy(h_hbm.at[pl.ds(s * RPW, RPW)],
                               h_sh.at[pl.ds(s * RPW, RPW)], g1)
        pltpu.sync_copy(src_hbm.at[w], src_v)
        pltpu.sync_copy(dst_hbm.at[w], dst_v)
        _fill_rows(buf0, CH, D, 0.0)

        @pl.loop(0, RPW // CH)
        def _(k):
            pltpu.sync_copy(buf0, acc_sh.at[pl.ds(s * RPW + k * CH, CH)])

        hst.wait()
        plsc.subcore_barrier()

        pltpu.async_copy(h_sh.at[src_v.at[0]], buf0, g0)

        @pl.loop(0, NCH, step=2)
        def _(j):
            pltpu.make_async_copy(h_sh.at[src_v.at[j]], buf0, g0).wait()
            sc0 = pltpu.async_copy(buf0, acc_sh.at[dst_v.at[j]], s0,
                                   add=True)
            gb1 = pltpu.async_copy(h_sh.at[src_v.at[j + 1]], buf1, g1)
            sc0.wait()
            gb1.wait()
            sc1 = pltpu.async_copy(buf1, acc_sh.at[dst_v.at[j + 1]], s1,
                                   add=True)

            @pl.when(j + 2 < NCH)
            def _():
                pltpu.async_copy(h_sh.at[src_v.at[j + 2]], buf0, g0)

            sc1.wait()

        plsc.subcore_barrier()
        pltpu.sync_copy(acc_sh.at[pl.ds(s * RPW, RPW)],
                        out_hbm.at[c, pl.ds(s * RPW, RPW)])

    return _agg


_agg_h = _make_agg(H)
_agg_c = _make_agg(CP)


RB = 1024
RBO = 2000


def _dinv(degp_ref):
    deg = degp_ref[0] + degp_ref[1] + 1.0
    return lax.rsqrt(deg)[:, None]


def _tc_mm1(x_ref, w1_ref, o_ref):
    o_ref[...] = jnp.dot(x_ref[...], w1_ref[...],
                         preferred_element_type=jnp.float32)


def _tc_scale(degp_ref, h_ref, o_ref):
    o_ref[...] = _dinv(degp_ref) * h_ref[...]


def _tc_mid(p_ref, hp_ref, degp_ref, w2_ref, b1_ref, o_ref):
    dinv = _dinv(degp_ref)
    a = dinv * (p_ref[0] + p_ref[1] + hp_ref[...]) + b1_ref[...]
    a = jnp.maximum(a, 0.0)
    o_ref[...] = dinv * jnp.dot(a, w2_ref[...],
                                preferred_element_type=jnp.float32)


def _tc_out(p_ref, hp_ref, degp_ref, b2_ref, o_ref):
    dinv = _dinv(degp_ref)
    o = dinv * (p_ref[0] + p_ref[1] + hp_ref[...]) + b2_ref[...]
    col = lax.broadcasted_iota(jnp.int32, o.shape, 1)
    valid = col < C
    neg = jnp.float32(-3.0e38)
    m = jnp.max(jnp.where(valid, o, neg), axis=1, keepdims=True)
    ssum = jnp.sum(jnp.where(valid, jnp.exp(o - m), 0.0), axis=1,
                   keepdims=True)
    o_ref[...] = (o - m - jnp.log(ssum))[:, :C]


def _bs(shape, imap):
    return pl.BlockSpec(shape, imap)


def _row(i):
    return (i, 0)


def _whole(i):
    return (0, 0)


def _mm1_call(xp, W1):
    return pl.pallas_call(
        _tc_mm1,
        out_shape=jax.ShapeDtypeStruct((NPAD, H), jnp.float32),
        grid=(NPAD // RB,),
        in_specs=[_bs((RB, F_IN), _row), _bs((F_IN, H), _whole)],
        out_specs=_bs((RB, H), _row),
    )(xp, W1)


def _scale_call(degp, h1):
    return pl.pallas_call(
        _tc_scale,
        out_shape=jax.ShapeDtypeStruct((NPAD, H), jnp.float32),
        grid=(NPAD // RB,),
        in_specs=[_bs((2, RB), lambda i: (0, i)), _bs((RB, H), _row)],
        out_specs=_bs((RB, H), _row),
    )(degp, h1)


def _mid_call(p1, h1p, degp, w2p, b1r):
    return pl.pallas_call(
        _tc_mid,
        out_shape=jax.ShapeDtypeStruct((NPAD, CP), jnp.float32),
        grid=(NPAD // RB,),
        in_specs=[
            _bs((2, RB, H), lambda i: (0, i, 0)),
            _bs((RB, H), _row),
            _bs((2, RB), lambda i: (0, i)),
            _bs((H, CP), _whole),
            _bs((1, H), _whole),
        ],
        out_specs=_bs((RB, CP), _row),
    )(p1, h1p, degp, w2p, b1r)


def _out_call(p2, h2p, degp, b2r):
    return pl.pallas_call(
        _tc_out,
        out_shape=jax.ShapeDtypeStruct((N, C), jnp.float32),
        grid=(NPAD // RB,),
        in_specs=[
            _bs((2, RB, CP), lambda i: (0, i, 0)),
            _bs((RB, CP), _row),
            _bs((2, RB), lambda i: (0, i)),
            _bs((1, CP), _whole),
        ],
        out_specs=_bs((RB, C), _row),
    )(p2, h2p, degp, b2r)


def kernel(x, edge_index, W1, b1, W2, b2):
    src = edge_index[0].astype(jnp.int32)
    dst = edge_index[1].astype(jnp.int32)
    pad = jnp.full((EPAD - E,), N, jnp.int32)
    srcp = jnp.concatenate([src, pad]).reshape(NW, NCH, CH)
    dstp = jnp.concatenate([dst, pad]).reshape(NW, NCH, CH)

    xp = jnp.pad(x, ((0, NPAD - N), (0, 0)))
    w2p = jnp.pad(W2, ((0, 0), (0, CP - C)))
    b1r = b1.reshape(1, H)
    b2r = jnp.pad(b2, (0, CP - C)).reshape(1, CP)

    degp = _deg_kernel(dstp)
    h1 = _mm1_call(xp, W1)
    h1p = _scale_call(degp, h1)
    p1 = _agg_h(h1p, srcp, dstp)
    h2p = _mid_call(p1, h1p, degp, w2p, b1r)
    p2 = _agg_c(h2p, srcp, dstp)
    return _out_call(p2, h2p, degp, b2r)


# speedup vs baseline: 46.6549x; 1.2367x over previous
import dataclasses
import functools

import jax
import jax.numpy as jnp
from jax import lax
from jax.experimental import pallas as pl
from jax.experimental.pallas import tpu as pltpu
from jax.experimental.pallas import tpu_sc as plsc

N = 10000
NPAD = 10240
F_IN = 128
H = 64
C = 40
CP = 64
E = 320000
NW = 32
EPW = 10240
CH = 128
NCH = EPW // CH
EPAD = NW * EPW
TPS = 16
RPW = NPAD // TPS
DEGW = 16

_mesh = plsc.VectorSubcoreMesh(core_axis_name="c", subcore_axis_name="s")
_sc_params = pltpu.CompilerParams(use_tc_tiling_on_sc=False)
if "needs_layout_passes" in pltpu.CompilerParams.__dataclass_fields__:
    _sc_params = dataclasses.replace(_sc_params, needs_layout_passes=False)


def _fill_rows(buf, nrows, width, value, dtype=jnp.float32, lanes=16):
    v = jnp.full((lanes,), value, dtype)

    @pl.loop(0, nrows)
    def _(r):
        for cc in range(0, width, lanes):
            buf[r, pl.ds(cc, lanes)] = v


@functools.partial(
    pl.kernel,
    out_type=jax.ShapeDtypeStruct((2, NPAD), jnp.float32),
    mesh=_mesh,
    compiler_params=_sc_params,
    scratch_types=[
        pltpu.VMEM((NCH, CH), jnp.int32),
        pltpu.VMEM((CH, DEGW), jnp.float32),
        pltpu.VMEM((RPW, DEGW), jnp.float32),
        pltpu.VMEM((RPW,), jnp.float32),
        pltpu.VMEM_SHARED((NPAD, DEGW), jnp.float32),
        pltpu.SemaphoreType.DMA,
    ],
)
def _deg_kernel(dst_hbm, out_hbm, dst_v, ones_v, reg_v, cmp_v, acc_sh, sem):
    c = lax.axis_index("c")
    s = lax.axis_index("s")
    w = c * TPS + s
    pltpu.sync_copy(dst_hbm.at[w], dst_v)
    _fill_rows(ones_v, CH, DEGW, 0.0)

    @pl.loop(0, RPW // CH)
    def _(k):
        pltpu.sync_copy(ones_v, acc_sh.at[pl.ds(s * RPW + k * CH, CH)])

    _fill_rows(ones_v, CH, DEGW, 1.0)
    plsc.subcore_barrier()

    @pl.loop(0, NCH, step=8)
    def _(j):
        for k in range(8):
            pltpu.async_copy(ones_v, acc_sh.at[dst_v.at[j + k]], sem,
                             add=True)
        for k in range(8):
            pltpu.make_async_copy(ones_v, acc_sh.at[dst_v.at[j + k]],
                                  sem).wait()

    plsc.subcore_barrier()
    pltpu.sync_copy(acc_sh.at[pl.ds(s * RPW, RPW)], reg_v)
    lane = jnp.arange(16, dtype=jnp.int32)
    zero16 = jnp.zeros((16,), jnp.int32)

    @pl.loop(0, RPW // 16)
    def _(j):
        vals = plsc.load_gather(reg_v, [j * 16 + lane, zero16])
        cmp_v[pl.ds(j * 16, 16)] = vals

    pltpu.sync_copy(cmp_v, out_hbm.at[c, pl.ds(s * RPW, RPW)])


def _make_agg(D):
    @functools.partial(
        pl.kernel,
        out_type=jax.ShapeDtypeStruct((2, NPAD, D), jnp.bfloat16),
        mesh=_mesh,
        compiler_params=_sc_params,
        scratch_types=[
            pltpu.VMEM((NCH, CH), jnp.int32),
            pltpu.VMEM((NCH, CH), jnp.int32),
            pltpu.VMEM((CH, D), jnp.bfloat16),
            pltpu.VMEM((CH, D), jnp.bfloat16),
            pltpu.VMEM_SHARED((NPAD, D), jnp.bfloat16),
            pltpu.VMEM_SHARED((NPAD, D), jnp.bfloat16),
            pltpu.SemaphoreType.DMA,
            pltpu.SemaphoreType.DMA,
            pltpu.SemaphoreType.DMA,
            pltpu.SemaphoreType.DMA,
        ],
    )
    def _agg(h_hbm, src_hbm, dst_hbm, out_hbm, src_v, dst_v, buf0, buf1,
             acc_sh, h_sh, g0, g1, s0, s1):
        c = lax.axis_index("c")
        s = lax.axis_index("s")
        w = c * TPS + s
        hst = pltpu.async_copy(h_hbm.at[pl.ds(s * RPW, RPW)],
                               h_sh.at[pl.ds(s * RPW, RPW)], g1)
        pltpu.sync_copy(src_hbm.at[w], src_v)
        pltpu.sync_copy(dst_hbm.at[w], dst_v)
        _fill_rows(buf0, CH, D, 0.0, jnp.bfloat16, 32)

        @pl.loop(0, RPW // CH)
        def _(k):
            pltpu.sync_copy(buf0, acc_sh.at[pl.ds(s * RPW + k * CH, CH)])

        hst.wait()
        plsc.subcore_barrier()

        pltpu.async_copy(h_sh.at[src_v.at[0]], buf0, g0)

        @pl.loop(0, NCH, step=2)
        def _(j):
            pltpu.make_async_copy(h_sh.at[src_v.at[j]], buf0, g0).wait()
            sc0 = pltpu.async_copy(buf0, acc_sh.at[dst_v.at[j]], s0,
                                   add=True)
            gb1 = pltpu.async_copy(h_sh.at[src_v.at[j + 1]], buf1, g1)
            sc0.wait()
            gb1.wait()
            sc1 = pltpu.async_copy(buf1, acc_sh.at[dst_v.at[j + 1]], s1,
                                   add=True)

            @pl.when(j + 2 < NCH)
            def _():
                pltpu.async_copy(h_sh.at[src_v.at[j + 2]], buf0, g0)

            sc1.wait()

        plsc.subcore_barrier()
        pltpu.sync_copy(acc_sh.at[pl.ds(s * RPW, RPW)],
                        out_hbm.at[c, pl.ds(s * RPW, RPW)])

    return _agg


_agg = _make_agg(H)


RB = 1024
RBO = 2000


def _dinv(degp_ref):
    deg = degp_ref[0] + degp_ref[1] + 1.0
    return lax.rsqrt(deg)[:, None]


def _tc_mm1(x_ref, w1_ref, o_ref):
    o_ref[...] = jnp.dot(x_ref[...], w1_ref[...],
                         preferred_element_type=jnp.float32)


def _tc_scale(degp_ref, h_ref, o_ref):
    o_ref[...] = (_dinv(degp_ref) * h_ref[...]).astype(jnp.bfloat16)


def _psum(p_ref, hp_ref):
    return (p_ref[0].astype(jnp.float32) + p_ref[1].astype(jnp.float32)
            + hp_ref[...].astype(jnp.float32))


def _tc_mid(p_ref, hp_ref, degp_ref, w2_ref, b1_ref, o_ref):
    dinv = _dinv(degp_ref)
    a = dinv * _psum(p_ref, hp_ref) + b1_ref[...]
    a = jnp.maximum(a, 0.0)
    o_ref[...] = (dinv * jnp.dot(a, w2_ref[...],
                                 preferred_element_type=jnp.float32)
                  ).astype(jnp.bfloat16)


def _tc_out(p_ref, hp_ref, degp_ref, b2_ref, o_ref):
    dinv = _dinv(degp_ref)
    o = dinv * _psum(p_ref, hp_ref) + b2_ref[...]
    col = lax.broadcasted_iota(jnp.int32, o.shape, 1)
    valid = col < C
    neg = jnp.float32(-3.0e38)
    m = jnp.max(jnp.where(valid, o, neg), axis=1, keepdims=True)
    ssum = jnp.sum(jnp.where(valid, jnp.exp(o - m), 0.0), axis=1,
                   keepdims=True)
    o_ref[...] = (o - m - jnp.log(ssum))[:, :C]


def _bs(shape, imap):
    return pl.BlockSpec(shape, imap)


def _row(i):
    return (i, 0)


def _whole(i):
    return (0, 0)


def _mm1_call(xp, W1):
    return pl.pallas_call(
        _tc_mm1,
        out_shape=jax.ShapeDtypeStruct((NPAD, H), jnp.float32),
        grid=(NPAD // RB,),
        in_specs=[_bs((RB, F_IN), _row), _bs((F_IN, H), _whole)],
        out_specs=_bs((RB, H), _row),
    )(xp, W1)


def _scale_call(degp, h1):
    return pl.pallas_call(
        _tc_scale,
        out_shape=jax.ShapeDtypeStruct((NPAD, H), jnp.bfloat16),
        grid=(NPAD // RB,),
        in_specs=[_bs((2, RB), lambda i: (0, i)), _bs((RB, H), _row)],
        out_specs=_bs((RB, H), _row),
    )(degp, h1)


def _mid_call(p1, h1p, degp, w2p, b1r):
    return pl.pallas_call(
        _tc_mid,
        out_shape=jax.ShapeDtypeStruct((NPAD, CP), jnp.bfloat16),
        grid=(NPAD // RB,),
        in_specs=[
            _bs((2, RB, H), lambda i: (0, i, 0)),
            _bs((RB, H), _row),
            _bs((2, RB), lambda i: (0, i)),
            _bs((H, CP), _whole),
            _bs((1, H), _whole),
        ],
        out_specs=_bs((RB, CP), _row),
    )(p1, h1p, degp, w2p, b1r)


def _out_call(p2, h2p, degp, b2r):
    return pl.pallas_call(
        _tc_out,
        out_shape=jax.ShapeDtypeStruct((N, C), jnp.float32),
        grid=(NPAD // RB,),
        in_specs=[
            _bs((2, RB, CP), lambda i: (0, i, 0)),
            _bs((RB, CP), _row),
            _bs((2, RB), lambda i: (0, i)),
            _bs((1, CP), _whole),
        ],
        out_specs=_bs((RB, C), _row),
    )(p2, h2p, degp, b2r)


def kernel(x, edge_index, W1, b1, W2, b2):
    src = edge_index[0].astype(jnp.int32)
    dst = edge_index[1].astype(jnp.int32)
    pad = jnp.full((EPAD - E,), N, jnp.int32)
    srcp = jnp.concatenate([src, pad]).reshape(NW, NCH, CH)
    dstp = jnp.concatenate([dst, pad]).reshape(NW, NCH, CH)

    xp = jnp.pad(x, ((0, NPAD - N), (0, 0)))
    w2p = jnp.pad(W2, ((0, 0), (0, CP - C)))
    b1r = b1.reshape(1, H)
    b2r = jnp.pad(b2, (0, CP - C)).reshape(1, CP)

    degp = _deg_kernel(dstp)
    h1 = _mm1_call(xp, W1)
    h1p = _scale_call(degp, h1)
    p1 = _agg(h1p, srcp, dstp)
    h2p = _mid_call(p1, h1p, degp, w2p, b1r)
    p2 = _agg(h2p, srcp, dstp)
    return _out_call(p2, h2p, degp, b2r)


# speedup vs baseline: 49.3250x; 1.0572x over previous
import dataclasses
import functools

import jax
import jax.numpy as jnp
from jax import lax
from jax.experimental import pallas as pl
from jax.experimental.pallas import tpu as pltpu
from jax.experimental.pallas import tpu_sc as plsc

N = 10000
NPAD = 10240
F_IN = 128
H = 64
C = 40
CP = 64
E = 320000
NW = 32
EPW = 10240
CH = 128
NCH = EPW // CH
EPAD = NW * EPW
TPS = 16
RPW = NPAD // TPS
DEGW = 16

_mesh = plsc.VectorSubcoreMesh(core_axis_name="c", subcore_axis_name="s")
_sc_params = pltpu.CompilerParams(use_tc_tiling_on_sc=False)
if "needs_layout_passes" in pltpu.CompilerParams.__dataclass_fields__:
    _sc_params = dataclasses.replace(_sc_params, needs_layout_passes=False)


def _fill_rows(buf, nrows, width, value, dtype=jnp.float32, lanes=16):
    v = jnp.full((lanes,), value, dtype)

    @pl.loop(0, nrows)
    def _(r):
        for cc in range(0, width, lanes):
            buf[r, pl.ds(cc, lanes)] = v


@functools.partial(
    pl.kernel,
    out_type=jax.ShapeDtypeStruct((2, NPAD), jnp.float32),
    mesh=_mesh,
    compiler_params=_sc_params,
    scratch_types=[
        pltpu.VMEM((NCH, CH), jnp.int32),
        pltpu.VMEM((CH, DEGW), jnp.float32),
        pltpu.VMEM((RPW, DEGW), jnp.float32),
        pltpu.VMEM((RPW,), jnp.float32),
        pltpu.VMEM_SHARED((NPAD, DEGW), jnp.float32),
        pltpu.SemaphoreType.DMA,
    ],
)
def _deg_kernel(edges_hbm, out_hbm, dst_v, ones_v, reg_v, cmp_v, acc_sh, sem):
    c = lax.axis_index("c")
    s = lax.axis_index("s")
    w = c * TPS + s
    pltpu.sync_copy(edges_hbm.at[1, w], dst_v)
    _fill_rows(ones_v, CH, DEGW, 0.0)

    @pl.loop(0, RPW // CH)
    def _(k):
        pltpu.sync_copy(ones_v, acc_sh.at[pl.ds(s * RPW + k * CH, CH)])

    _fill_rows(ones_v, CH, DEGW, 1.0)
    plsc.subcore_barrier()

    @pl.loop(0, NCH, step=8)
    def _(j):
        for k in range(8):
            pltpu.async_copy(ones_v, acc_sh.at[dst_v.at[j + k]], sem,
                             add=True)
        for k in range(8):
            pltpu.make_async_copy(ones_v, acc_sh.at[dst_v.at[j + k]],
                                  sem).wait()

    plsc.subcore_barrier()
    pltpu.sync_copy(acc_sh.at[pl.ds(s * RPW, RPW)], reg_v)
    lane = jnp.arange(16, dtype=jnp.int32)
    zero16 = jnp.zeros((16,), jnp.int32)

    @pl.loop(0, RPW // 16)
    def _(j):
        vals = plsc.load_gather(reg_v, [j * 16 + lane, zero16])
        cmp_v[pl.ds(j * 16, 16)] = vals

    pltpu.sync_copy(cmp_v, out_hbm.at[c, pl.ds(s * RPW, RPW)])


def _make_agg(D):
    @functools.partial(
        pl.kernel,
        out_type=jax.ShapeDtypeStruct((2, NPAD, D), jnp.bfloat16),
        mesh=_mesh,
        compiler_params=_sc_params,
        scratch_types=[
            pltpu.VMEM((NCH, CH), jnp.int32),
            pltpu.VMEM((NCH, CH), jnp.int32),
            pltpu.VMEM((CH, D), jnp.bfloat16),
            pltpu.VMEM((CH, D), jnp.bfloat16),
            pltpu.VMEM_SHARED((NPAD, D), jnp.bfloat16),
            pltpu.VMEM_SHARED((NPAD, D), jnp.bfloat16),
            pltpu.SemaphoreType.DMA,
            pltpu.SemaphoreType.DMA,
            pltpu.SemaphoreType.DMA,
            pltpu.SemaphoreType.DMA,
        ],
    )
    def _agg(h_hbm, edges_hbm, out_hbm, src_v, dst_v, buf0, buf1,
             acc_sh, h_sh, g0, g1, s0, s1):
        c = lax.axis_index("c")
        s = lax.axis_index("s")
        w = c * TPS + s
        hst = pltpu.async_copy(h_hbm.at[pl.ds(s * RPW, RPW)],
                               h_sh.at[pl.ds(s * RPW, RPW)], g1)
        pltpu.sync_copy(edges_hbm.at[0, w], src_v)
        pltpu.sync_copy(edges_hbm.at[1, w], dst_v)
        _fill_rows(buf0, CH, D, 0.0, jnp.bfloat16, 32)

        @pl.loop(0, RPW // CH)
        def _(k):
            pltpu.sync_copy(buf0, acc_sh.at[pl.ds(s * RPW + k * CH, CH)])

        hst.wait()
        plsc.subcore_barrier()

        pltpu.async_copy(h_sh.at[src_v.at[0]], buf0, g0)

        @pl.loop(0, NCH, step=2)
        def _(j):
            pltpu.make_async_copy(h_sh.at[src_v.at[j]], buf0, g0).wait()
            sc0 = pltpu.async_copy(buf0, acc_sh.at[dst_v.at[j]], s0,
                                   add=True)
            gb1 = pltpu.async_copy(h_sh.at[src_v.at[j + 1]], buf1, g1)
            sc0.wait()
            gb1.wait()
            sc1 = pltpu.async_copy(buf1, acc_sh.at[dst_v.at[j + 1]], s1,
                                   add=True)

            @pl.when(j + 2 < NCH)
            def _():
                pltpu.async_copy(h_sh.at[src_v.at[j + 2]], buf0, g0)

            sc1.wait()

        plsc.subcore_barrier()
        pltpu.sync_copy(acc_sh.at[pl.ds(s * RPW, RPW)],
                        out_hbm.at[c, pl.ds(s * RPW, RPW)])

    return _agg


_agg = _make_agg(H)


RB = 1024
RBO = 2000


def _dinv(degp_ref):
    deg = degp_ref[0] + degp_ref[1] + 1.0
    return lax.rsqrt(deg)[:, None]


def _tc_mm1(x_ref, w1_ref, o_ref):
    o_ref[...] = jnp.dot(x_ref[...], w1_ref[...],
                         preferred_element_type=jnp.float32)


def _tc_prep(e_ref, o_ref):
    e = e_ref[...].reshape(2, E // CH, CH)
    padrows = jnp.full((2, EPAD // CH - E // CH, CH), N, jnp.int32)
    o_ref[...] = jnp.concatenate([e, padrows], axis=1).reshape(
        2, NW, NCH, CH)


def _tc_scale(degp_ref, h_ref, o_ref):
    o_ref[...] = (_dinv(degp_ref) * h_ref[...]).astype(jnp.bfloat16)


def _psum(p_ref, hp_ref):
    return (p_ref[0].astype(jnp.float32) + p_ref[1].astype(jnp.float32)
            + hp_ref[...].astype(jnp.float32))


def _tc_mid(p_ref, hp_ref, degp_ref, w2_ref, b1_ref, o_ref):
    dinv = _dinv(degp_ref)
    a = dinv * _psum(p_ref, hp_ref) + b1_ref[...]
    a = jnp.maximum(a, 0.0)
    o_ref[...] = (dinv * jnp.dot(a, w2_ref[...],
                                 preferred_element_type=jnp.float32)
                  ).astype(jnp.bfloat16)


def _tc_out(p_ref, hp_ref, degp_ref, b2_ref, o_ref):
    dinv = _dinv(degp_ref)
    o = dinv * _psum(p_ref, hp_ref) + b2_ref[...]
    col = lax.broadcasted_iota(jnp.int32, o.shape, 1)
    valid = col < C
    neg = jnp.float32(-3.0e38)
    m = jnp.max(jnp.where(valid, o, neg), axis=1, keepdims=True)
    ssum = jnp.sum(jnp.where(valid, jnp.exp(o - m), 0.0), axis=1,
                   keepdims=True)
    o_ref[...] = (o - m - jnp.log(ssum))[:, :C]


def _bs(shape, imap):
    return pl.BlockSpec(shape, imap)


def _row(i):
    return (i, 0)


def _whole(i):
    return (0, 0)


def _mm1_call(xp, W1):
    return pl.pallas_call(
        _tc_mm1,
        out_shape=jax.ShapeDtypeStruct((NPAD, H), jnp.float32),
        grid=(NPAD // RB,),
        in_specs=[_bs((RB, F_IN), _row), _bs((F_IN, H), _whole)],
        out_specs=_bs((RB, H), _row),
    )(xp, W1)


def _scale_call(degp, h1):
    return pl.pallas_call(
        _tc_scale,
        out_shape=jax.ShapeDtypeStruct((NPAD, H), jnp.bfloat16),
        grid=(NPAD // RB,),
        in_specs=[_bs((2, RB), lambda i: (0, i)), _bs((RB, H), _row)],
        out_specs=_bs((RB, H), _row),
    )(degp, h1)


def _mid_call(p1, h1p, degp, w2p, b1r):
    return pl.pallas_call(
        _tc_mid,
        out_shape=jax.ShapeDtypeStruct((NPAD, CP), jnp.bfloat16),
        grid=(NPAD // RB,),
        in_specs=[
            _bs((2, RB, H), lambda i: (0, i, 0)),
            _bs((RB, H), _row),
            _bs((2, RB), lambda i: (0, i)),
            _bs((H, CP), _whole),
            _bs((1, H), _whole),
        ],
        out_specs=_bs((RB, CP), _row),
    )(p1, h1p, degp, w2p, b1r)


def _out_call(p2, h2p, degp, b2r):
    return pl.pallas_call(
        _tc_out,
        out_shape=jax.ShapeDtypeStruct((N, C), jnp.float32),
        grid=(NPAD // RB,),
        in_specs=[
            _bs((2, RB, CP), lambda i: (0, i, 0)),
            _bs((RB, CP), _row),
            _bs((2, RB), lambda i: (0, i)),
            _bs((1, CP), _whole),
        ],
        out_specs=_bs((RB, C), _row),
    )(p2, h2p, degp, b2r)


def kernel(x, edge_index, W1, b1, W2, b2):
    edges = pl.pallas_call(
        _tc_prep,
        out_shape=jax.ShapeDtypeStruct((2, NW, NCH, CH), jnp.int32),
    )(edge_index.astype(jnp.int32))

    xp = jnp.pad(x, ((0, NPAD - N), (0, 0)))
    w2p = jnp.pad(W2, ((0, 0), (0, CP - C)))
    b1r = b1.reshape(1, H)
    b2r = jnp.pad(b2, (0, CP - C)).reshape(1, CP)

    degp = _deg_kernel(edges)
    h1 = _mm1_call(xp, W1)
    h1p = _scale_call(degp, h1)
    p1 = _agg(h1p, edges)
    h2p = _mid_call(p1, h1p, degp, w2p, b1r)
    p2 = _agg(h2p, edges)
    return _out_call(p2, h2p, degp, b2r)


# speedup vs baseline: 50.6589x; 1.0270x over previous
import dataclasses
import functools

import jax
import jax.numpy as jnp
from jax import lax
from jax.experimental import pallas as pl
from jax.experimental.pallas import tpu as pltpu
from jax.experimental.pallas import tpu_sc as plsc

N = 10000
NPAD = 10240
F_IN = 128
H = 64
C = 40
CP = 64
E = 320000
NW = 32
EPW = 10240
CH = 128
NCH = EPW // CH
EPAD = NW * EPW
TPS = 16
RPW = NPAD // TPS
DEGW = 16

_mesh = plsc.VectorSubcoreMesh(core_axis_name="c", subcore_axis_name="s")
_sc_params = pltpu.CompilerParams(use_tc_tiling_on_sc=False)
if "needs_layout_passes" in pltpu.CompilerParams.__dataclass_fields__:
    _sc_params = dataclasses.replace(_sc_params, needs_layout_passes=False)


def _fill_rows(buf, nrows, width, value, dtype=jnp.float32, lanes=16):
    v = jnp.full((lanes,), value, dtype)

    @pl.loop(0, nrows)
    def _(r):
        for cc in range(0, width, lanes):
            buf[r, pl.ds(cc, lanes)] = v


@functools.partial(
    pl.kernel,
    out_type=jax.ShapeDtypeStruct((2, 2, NPAD // 2), jnp.float32),
    mesh=_mesh,
    compiler_params=_sc_params,
    scratch_types=[
        pltpu.VMEM((NCH, CH), jnp.int32),
        pltpu.VMEM((CH, DEGW), jnp.float32),
        pltpu.VMEM((RPW, DEGW), jnp.float32),
        pltpu.VMEM((RPW // 2,), jnp.float32),
        pltpu.VMEM((RPW // 2,), jnp.float32),
        pltpu.VMEM_SHARED((NPAD, DEGW), jnp.float32),
        pltpu.SemaphoreType.DMA,
    ],
)
def _deg_kernel(edges_hbm, out_hbm, dst_v, ones_v, reg_v, cmp_lo, cmp_hi,
                acc_sh, sem):
    c = lax.axis_index("c")
    s = lax.axis_index("s")
    w = c * TPS + s
    pltpu.sync_copy(edges_hbm.at[1, w], dst_v)
    _fill_rows(ones_v, CH, DEGW, 0.0)

    @pl.loop(0, RPW // CH)
    def _(k):
        pltpu.sync_copy(ones_v, acc_sh.at[pl.ds(s * RPW + k * CH, CH)])

    _fill_rows(ones_v, CH, DEGW, 1.0)
    plsc.subcore_barrier()

    @pl.loop(0, NCH, step=8)
    def _(j):
        for k in range(8):
            pltpu.async_copy(ones_v, acc_sh.at[dst_v.at[j + k]], sem,
                             add=True)
        for k in range(8):
            pltpu.make_async_copy(ones_v, acc_sh.at[dst_v.at[j + k]],
                                  sem).wait()

    plsc.subcore_barrier()
    pltpu.sync_copy(acc_sh.at[pl.ds(s * RPW, RPW)], reg_v)
    lane = jnp.arange(16, dtype=jnp.int32)
    zero16 = jnp.zeros((16,), jnp.int32)

    @pl.loop(0, RPW // 32)
    def _(j):
        ve = plsc.load_gather(reg_v, [j * 32 + lane * 2, zero16])
        vo = plsc.load_gather(reg_v, [j * 32 + lane * 2 + 1, zero16])
        cmp_lo[pl.ds(j * 16, 16)] = ve
        cmp_hi[pl.ds(j * 16, 16)] = vo

    pltpu.sync_copy(cmp_lo, out_hbm.at[c, 0, pl.ds(s * (RPW // 2), RPW // 2)])
    pltpu.sync_copy(cmp_hi, out_hbm.at[c, 1, pl.ds(s * (RPW // 2), RPW // 2)])


def _make_agg(D):
    @functools.partial(
        pl.kernel,
        out_type=jax.ShapeDtypeStruct((2, NPAD, D), jnp.bfloat16),
        mesh=_mesh,
        compiler_params=_sc_params,
        scratch_types=[
            pltpu.VMEM((NCH, CH), jnp.int32),
            pltpu.VMEM((NCH, CH), jnp.int32),
            pltpu.VMEM((CH, D), jnp.bfloat16),
            pltpu.VMEM((CH, D), jnp.bfloat16),
            pltpu.VMEM_SHARED((NPAD, D), jnp.bfloat16),
            pltpu.VMEM_SHARED((NPAD, D), jnp.bfloat16),
            pltpu.SemaphoreType.DMA,
            pltpu.SemaphoreType.DMA,
            pltpu.SemaphoreType.DMA,
            pltpu.SemaphoreType.DMA,
        ],
    )
    def _agg(h_hbm, edges_hbm, out_hbm, src_v, dst_v, buf0, buf1,
             acc_sh, h_sh, g0, g1, s0, s1):
        c = lax.axis_index("c")
        s = lax.axis_index("s")
        w = c * TPS + s
        hst = pltpu.async_copy(h_hbm.at[pl.ds(s * RPW, RPW)],
                               h_sh.at[pl.ds(s * RPW, RPW)], g1)
        pltpu.sync_copy(edges_hbm.at[0, w], src_v)
        pltpu.sync_copy(edges_hbm.at[1, w], dst_v)
        _fill_rows(buf0, CH, D, 0.0, jnp.bfloat16, 32)

        @pl.loop(0, RPW // CH)
        def _(k):
            pltpu.sync_copy(buf0, acc_sh.at[pl.ds(s * RPW + k * CH, CH)])

        hst.wait()
        plsc.subcore_barrier()

        pltpu.async_copy(h_sh.at[src_v.at[0]], buf0, g0)

        @pl.loop(0, NCH, step=2)
        def _(j):
            pltpu.make_async_copy(h_sh.at[src_v.at[j]], buf0, g0).wait()
            sc0 = pltpu.async_copy(buf0, acc_sh.at[dst_v.at[j]], s0,
                                   add=True)
            gb1 = pltpu.async_copy(h_sh.at[src_v.at[j + 1]], buf1, g1)
            sc0.wait()
            gb1.wait()
            sc1 = pltpu.async_copy(buf1, acc_sh.at[dst_v.at[j + 1]], s1,
                                   add=True)

            @pl.when(j + 2 < NCH)
            def _():
                pltpu.async_copy(h_sh.at[src_v.at[j + 2]], buf0, g0)

            sc1.wait()

        plsc.subcore_barrier()
        pltpu.sync_copy(acc_sh.at[pl.ds(s * RPW, RPW)],
                        out_hbm.at[c, pl.ds(s * RPW, RPW)])

    return _agg


_agg = _make_agg(H)


RB = 1024
RBO = 2000


def _dinvp(degp_ref):
    deg = degp_ref[0] + degp_ref[1] + 1.0
    dinv = lax.rsqrt(deg)
    lo = jnp.broadcast_to(dinv[0][:, None], (RB // 2, H))
    hi = jnp.broadcast_to(dinv[1][:, None], (RB // 2, H))
    return jnp.concatenate([lo, hi], axis=1)


def _tc_mm1(x_ref, w1_ref, o_ref):
    o_ref[...] = jnp.dot(x_ref[...], w1_ref[...],
                         preferred_element_type=jnp.float32)


def _tc_prep(e_ref, o_ref):
    e = e_ref[...].reshape(2, E // CH, CH)
    padrows = jnp.full((2, EPAD // CH - E // CH, CH), N, jnp.int32)
    e = jnp.concatenate([e, padrows], axis=1)
    e = jnp.where(e < NPAD // 2, 2 * e, 2 * e - (NPAD - 1))
    o_ref[...] = e.reshape(2, NW, NCH, CH)


def _tc_scale(degp_ref, hlo_ref, hhi_ref, o_ref):
    hq = jnp.concatenate([hlo_ref[...], hhi_ref[...]], axis=1)
    o_ref[...] = (_dinvp(degp_ref) * hq).astype(jnp.bfloat16)


def _psum(p_ref, hp_ref):
    return (p_ref[0].astype(jnp.float32) + p_ref[1].astype(jnp.float32)
            + hp_ref[...].astype(jnp.float32))


def _tc_mid(p_ref, hp_ref, degp_ref, w2_ref, b1_ref, o_ref):
    dp = _dinvp(degp_ref)
    a = jnp.maximum(dp * _psum(p_ref, hp_ref) + b1_ref[...], 0.0)
    h2 = jnp.dot(a, w2_ref[...], preferred_element_type=jnp.float32)
    o_ref[...] = (dp * h2).astype(jnp.bfloat16)


def _tc_out(p_ref, hp_ref, degp_ref, b2_ref, olo_ref, ohi_ref):
    o = _dinvp(degp_ref) * _psum(p_ref, hp_ref) + b2_ref[...]
    neg = jnp.float32(-3.0e38)
    for k, out_ref in ((0, olo_ref), (1, ohi_ref)):
        oh = o[:, k * H:(k + 1) * H]
        col = lax.broadcasted_iota(jnp.int32, oh.shape, 1)
        valid = col < C
        m = jnp.max(jnp.where(valid, oh, neg), axis=1, keepdims=True)
        ssum = jnp.sum(jnp.where(valid, jnp.exp(oh - m), 0.0), axis=1,
                       keepdims=True)
        out_ref[...] = (oh - m - jnp.log(ssum))[:, :C]


def _bs(shape, imap):
    return pl.BlockSpec(shape, imap)


def _row(i):
    return (i, 0)


def _whole(i):
    return (0, 0)


def _mm1_call(xp, W1):
    return pl.pallas_call(
        _tc_mm1,
        out_shape=jax.ShapeDtypeStruct((NPAD, H), jnp.float32),
        grid=(NPAD // RB,),
        in_specs=[_bs((RB, F_IN), _row), _bs((F_IN, H), _whole)],
        out_specs=_bs((RB, H), _row),
    )(xp, W1)


_HB = NPAD // 2 // (RB // 2)


def _scale_call(degp, h1):
    return pl.pallas_call(
        _tc_scale,
        out_shape=jax.ShapeDtypeStruct((NPAD // 2, 128), jnp.bfloat16),
        grid=(NPAD // RB,),
        in_specs=[
            _bs((2, 2, RB // 2), lambda i: (0, 0, i)),
            _bs((RB // 2, H), _row),
            _bs((RB // 2, H), lambda i: (i + _HB, 0)),
        ],
        out_specs=_bs((RB // 2, 128), _row),
    )(degp, h1, h1)


def _mid_call(p1, h1p, degp, w2bd, b1q):
    return pl.pallas_call(
        _tc_mid,
        out_shape=jax.ShapeDtypeStruct((NPAD // 2, 128), jnp.bfloat16),
        grid=(NPAD // RB,),
        in_specs=[
            _bs((2, RB // 2, 128), lambda i: (0, i, 0)),
            _bs((RB // 2, 128), _row),
            _bs((2, 2, RB // 2), lambda i: (0, 0, i)),
            _bs((128, 128), _whole),
            _bs((1, 128), _whole),
        ],
        out_specs=_bs((RB // 2, 128), _row),
    )(p1, h1p, degp, w2bd, b1q)


def _out_call(p2, h2p, degp, b2q):
    return pl.pallas_call(
        _tc_out,
        out_shape=[
            jax.ShapeDtypeStruct((NPAD // 2, C), jnp.float32),
            jax.ShapeDtypeStruct((NPAD // 2, C), jnp.float32),
        ],
        grid=(NPAD // RB,),
        in_specs=[
            _bs((2, RB // 2, 128), lambda i: (0, i, 0)),
            _bs((RB // 2, 128), _row),
            _bs((2, 2, RB // 2), lambda i: (0, 0, i)),
            _bs((1, 128), _whole),
        ],
        out_specs=[_bs((RB // 2, C), _row), _bs((RB // 2, C), _row)],
    )(p2, h2p, degp, b2q)


def kernel(x, edge_index, W1, b1, W2, b2):
    edges = pl.pallas_call(
        _tc_prep,
        out_shape=jax.ShapeDtypeStruct((2, NW, NCH, CH), jnp.int32),
    )(edge_index.astype(jnp.int32))

    xp = jnp.pad(x, ((0, NPAD - N), (0, 0)))
    w2p = jnp.pad(W2, ((0, 0), (0, CP - C)))
    z64 = jnp.zeros((H, H), jnp.float32)
    w2bd = jnp.concatenate(
        [jnp.concatenate([w2p, z64], axis=1),
         jnp.concatenate([z64, w2p], axis=1)], axis=0)
    b1q = jnp.concatenate([b1, b1]).reshape(1, 128)
    b2p = jnp.pad(b2, (0, CP - C))
    b2q = jnp.concatenate([b2p, b2p]).reshape(1, 128)

    degp = _deg_kernel(edges)
    h1 = _mm1_call(xp, W1)
    h1p = _scale_call(degp, h1)
    p1 = _agg(h1p.reshape(NPAD, H), edges)
    h2p = _mid_call(p1.reshape(2, NPAD // 2, 128), h1p, degp, w2bd, b1q)
    p2 = _agg(h2p.reshape(NPAD, CP), edges)
    out_lo, out_hi = _out_call(p2.reshape(2, NPAD // 2, 128), h2p, degp,
                               b2q)
    return jnp.concatenate([out_lo, out_hi[:N - NPAD // 2]], axis=0)
